# baseline (speedup 1.0000x reference)
import functools
import jax
import jax.numpy as jnp
from jax import lax
from jax.experimental import pallas as pl
from jax.experimental.pallas import tpu as pltpu
from jax.experimental.pallas import tpu_sc as plsc


def _prep_kernel(c_ref, srt_ref, mid_ref):
    c = c_ref[...]
    k = c.shape[1]
    ci = c[:, :, None]
    cj = c[:, None, :]
    ii = lax.broadcasted_iota(jnp.int32, (1, k, k), 1)
    jj = lax.broadcasted_iota(jnp.int32, (1, k, k), 2)
    rank = jnp.sum(
        jnp.where((cj < ci) | ((cj == ci) & (jj < ii)), 1, 0), axis=2)
    rr = lax.broadcasted_iota(jnp.int32, (1, k, k), 1)
    oh = (rank[:, None, :] == rr).astype(c.dtype)
    srt = jnp.sum(oh * c[:, None, :], axis=2)
    nxt = jnp.concatenate([srt[:, 1:], srt[:, k - 1:]], axis=1)
    srt_ref[...] = srt
    mid_ref[...] = 0.5 * (srt + nxt)


_ROWS = 32768
_NW = 32
_ROWS_PER_W = _ROWS // _NW
_CH_ROWS = 256
_N_CHUNKS = _ROWS_PER_W // _CH_ROWS


def _sc_kernel(k, n_mels, x_hbm, srt_hbm, mid_hbm, out_hbm,
               srt_v, mid_v, xbuf, obuf):
    wid = lax.axis_index("s") * 2 + lax.axis_index("c")
    base_row = wid * _ROWS_PER_W
    pltpu.sync_copy(srt_hbm, srt_v)
    pltpu.sync_copy(mid_hbm, mid_v)
    iota16 = lax.broadcasted_iota(jnp.int32, (16,), 0)
    bases = [(iota16 + 16 * p) * k for p in range(n_mels // 16)]

    def chunk_body(ci, carry):
        row0 = base_row + ci * _CH_ROWS
        pltpu.sync_copy(x_hbm.at[pl.ds(row0, _CH_ROWS)], xbuf)

        def row_body(r, carry2):
            for p, base in enumerate(bases):
                x = xbuf[r, pl.ds(16 * p, 16)]
                pos = jnp.zeros((16,), jnp.int32)
                add = k // 2
                while add >= 1:
                    node = base + pos + (add - 1)
                    mval = plsc.load_gather(mid_v, [node])
                    pos = pos + jnp.where(x > mval, add, 0)
                    add //= 2
                obuf[r, pl.ds(16 * p, 16)] = plsc.load_gather(
                    srt_v, [base + pos])
            return carry2

        lax.fori_loop(0, _CH_ROWS, row_body, 0)
        pltpu.sync_copy(obuf, out_hbm.at[pl.ds(row0, _CH_ROWS)])
        return carry

    lax.fori_loop(0, _N_CHUNKS, chunk_body, 0)


def kernel(melspecs, centroids):
    b, t, n_mels = melspecs.shape
    k = centroids.shape[1]
    srt, mid = pl.pallas_call(
        _prep_kernel,
        out_shape=[
            jax.ShapeDtypeStruct((n_mels, k), centroids.dtype),
            jax.ShapeDtypeStruct((n_mels, k), centroids.dtype),
        ],
    )(centroids)
    x2 = melspecs.reshape(_ROWS, n_mels)
    sc = functools.partial(
        pl.kernel,
        mesh=plsc.VectorSubcoreMesh(core_axis_name="c", subcore_axis_name="s"),
        compiler_params=pltpu.CompilerParams(needs_layout_passes=False),
        out_type=jax.ShapeDtypeStruct((_ROWS, n_mels), melspecs.dtype),
        scratch_types=[
            pltpu.VMEM((n_mels * k,), melspecs.dtype),
            pltpu.VMEM((n_mels * k,), melspecs.dtype),
            pltpu.VMEM((_CH_ROWS, n_mels), melspecs.dtype),
            pltpu.VMEM((_CH_ROWS, n_mels), melspecs.dtype),
        ],
    )(functools.partial(_sc_kernel, k, n_mels))
    out = sc(x2, srt.reshape(-1), mid.reshape(-1))
    return out.reshape(b, t, n_mels)


# speedup vs baseline: 2.5314x; 2.5314x over previous
import functools
import jax
import jax.numpy as jnp
from jax import lax
from jax.experimental import pallas as pl
from jax.experimental.pallas import tpu as pltpu
from jax.experimental.pallas import tpu_sc as plsc


def _prep_kernel(c_ref, srt_ref, mid_ref):
    c = c_ref[...]
    k = c.shape[1]
    ci = c[:, :, None]
    cj = c[:, None, :]
    ii = lax.broadcasted_iota(jnp.int32, (1, k, k), 1)
    jj = lax.broadcasted_iota(jnp.int32, (1, k, k), 2)
    rank = jnp.sum(
        jnp.where((cj < ci) | ((cj == ci) & (jj < ii)), 1, 0), axis=2)
    rr = lax.broadcasted_iota(jnp.int32, (1, k, k), 1)
    oh = (rank[:, None, :] == rr).astype(c.dtype)
    srt = jnp.sum(oh * c[:, None, :], axis=2)
    nxt = jnp.concatenate([srt[:, 1:], srt[:, k - 1:]], axis=1)
    srt_ref[...] = srt
    mid_ref[...] = 0.5 * (srt + nxt)


_ROWS = 32768
_NW = 32
_ROWS_PER_W = _ROWS // _NW
_CH_ROWS = 256
_N_CHUNKS = _ROWS_PER_W // _CH_ROWS


def _sc_kernel(k, n_mels, x_hbm, srt_hbm, mid_hbm, out_hbm,
               srt_v, mid_v, xbuf, obuf):
    wid = lax.axis_index("s") * 2 + lax.axis_index("c")
    base_row = wid * _ROWS_PER_W
    pltpu.sync_copy(srt_hbm, srt_v)
    pltpu.sync_copy(mid_hbm, mid_v)
    iota16 = lax.broadcasted_iota(jnp.int32, (16,), 0)
    bases = [(iota16 + 16 * p) * k for p in range(n_mels // 16)]

    def chunk_body(ci, carry):
        row0 = base_row + ci * _CH_ROWS
        pltpu.sync_copy(x_hbm.at[pl.ds(row0, _CH_ROWS)], xbuf)

        @plsc.parallel_loop(0, _CH_ROWS, unroll=4)
        def row_body(r):
            for p, base in enumerate(bases):
                x = xbuf[r, pl.ds(16 * p, 16)]
                pos = jnp.zeros((16,), jnp.int32)
                add = k // 2
                while add >= 1:
                    node = base + pos + (add - 1)
                    mval = plsc.load_gather(mid_v, [node])
                    pos = pos + jnp.where(x > mval, add, 0)
                    add //= 2
                obuf[r, pl.ds(16 * p, 16)] = plsc.load_gather(
                    srt_v, [base + pos])
        pltpu.sync_copy(obuf, out_hbm.at[pl.ds(row0, _CH_ROWS)])
        return carry

    lax.fori_loop(0, _N_CHUNKS, chunk_body, 0)


def kernel(melspecs, centroids):
    b, t, n_mels = melspecs.shape
    k = centroids.shape[1]
    srt, mid = pl.pallas_call(
        _prep_kernel,
        out_shape=[
            jax.ShapeDtypeStruct((n_mels, k), centroids.dtype),
            jax.ShapeDtypeStruct((n_mels, k), centroids.dtype),
        ],
    )(centroids)
    x2 = melspecs.reshape(_ROWS, n_mels)
    sc = functools.partial(
        pl.kernel,
        mesh=plsc.VectorSubcoreMesh(core_axis_name="c", subcore_axis_name="s"),
        compiler_params=pltpu.CompilerParams(needs_layout_passes=False),
        out_type=jax.ShapeDtypeStruct((_ROWS, n_mels), melspecs.dtype),
        scratch_types=[
            pltpu.VMEM((n_mels * k,), melspecs.dtype),
            pltpu.VMEM((n_mels * k,), melspecs.dtype),
            pltpu.VMEM((_CH_ROWS, n_mels), melspecs.dtype),
            pltpu.VMEM((_CH_ROWS, n_mels), melspecs.dtype),
        ],
    )(functools.partial(_sc_kernel, k, n_mels))
    out = sc(x2, srt.reshape(-1), mid.reshape(-1))
    return out.reshape(b, t, n_mels)
